# baseline (speedup 1.0000x reference)
import functools

import jax
import jax.numpy as jnp
from jax import lax
from jax.experimental import pallas as pl
from jax.experimental.pallas import tpu as pltpu
from jax.experimental.pallas import tpu_sc as plsc

D = 64
SCALE = 8.0

NC = 2
NS = 16
NW = NC * NS

BATCH = 4096
SEQ = 200
B = BATCH * SEQ
IDX_MINOR = 128
ROWS_PER_W = B // (NW * IDX_MINOR)
R = 2
CHUNK = R * IDX_MINOR
NCHUNK = ROWS_PER_W // R
NBUF = 4


@functools.partial(
    pl.kernel,
    out_type=jax.ShapeDtypeStruct((B, D), jnp.float32),
    mesh=plsc.VectorSubcoreMesh(core_axis_name="c", subcore_axis_name="s"),
    compiler_params=pltpu.CompilerParams(use_tc_tiling_on_sc=False),
    scratch_types=[
        pltpu.VMEM((ROWS_PER_W, IDX_MINOR), jnp.int32),
        pltpu.VMEM((CHUNK, D), jnp.float32),
        pltpu.VMEM((CHUNK, D), jnp.float32),
        pltpu.VMEM((CHUNK, D), jnp.float32),
        pltpu.VMEM((CHUNK, D), jnp.float32),
        pltpu.SemaphoreType.DMA,
        pltpu.SemaphoreType.DMA,
        pltpu.SemaphoreType.DMA,
        pltpu.SemaphoreType.DMA,
        pltpu.SemaphoreType.DMA,
        pltpu.SemaphoreType.DMA,
        pltpu.SemaphoreType.DMA,
        pltpu.SemaphoreType.DMA,
    ],
)
def _sc_gather(table_hbm, idx_hbm, out_hbm,
               idx_v, r0, r1, r2, r3, g0, g1, g2, g3, w0, w1, w2, w3):
    rows = (r0, r1, r2, r3)
    gsem = (g0, g1, g2, g3)
    wsem = (w0, w1, w2, w3)

    wid = lax.axis_index("s") * NC + lax.axis_index("c")
    rbase = wid * ROWS_PER_W
    obase = wid * (ROWS_PER_W * IDX_MINOR)

    pltpu.sync_copy(idx_hbm.at[pl.ds(rbase, ROWS_PER_W)], idx_v)

    def fire(c, b):
        for j in range(R):
            pltpu.async_copy(
                table_hbm.at[idx_v.at[c * R + j]],
                rows[b].at[pl.ds(j * IDX_MINOR, IDX_MINOR)],
                gsem[b])

    def drain(c, b):
        for j in range(R):
            pltpu.make_async_copy(
                table_hbm.at[idx_v.at[c * R + j]],
                rows[b].at[pl.ds(j * IDX_MINOR, IDX_MINOR)],
                gsem[b]).wait()

    def write(c, b):
        pltpu.async_copy(
            rows[b], out_hbm.at[pl.ds(obase + c * CHUNK, CHUNK)], wsem[b])

    def wait_write(c, b):
        pltpu.make_async_copy(
            rows[b], out_hbm.at[pl.ds(obase + c * CHUNK, CHUNK)], wsem[b]).wait()

    for c in range(NBUF - 1):
        fire(c, c)

    def step(p, carry):
        for b in range(NBUF):
            c = p * NBUF + b
            drain(c, b)
            write(c, b)
            fb = (b + NBUF - 1) % NBUF
            fc = c + NBUF - 1

            @pl.when(jnp.logical_and(fc >= NBUF, fc < NCHUNK))
            def _():
                wait_write(fc - NBUF, fb)

            @pl.when(fc < NCHUNK)
            def _():
                fire(fc, fb)
        return carry

    lax.fori_loop(0, NCHUNK // NBUF, step, 0)

    for b in range(NBUF):
        wait_write(NCHUNK - NBUF + b, b)


def _finish_block(x_ref, o_ref):
    x = x_ref[...]
    for i in range(8):
        o_ref[i, :, 0] = (
            jnp.transpose(x[:, i, :]) * SCALE).reshape(8, 8, IDX_MINOR)


def _tc_finish(flat):
    x3 = flat.reshape(BATCH, SEQ, D)
    return pl.pallas_call(
        _finish_block,
        grid=(NW, SEQ // 8),
        in_specs=[pl.BlockSpec((IDX_MINOR, 8, D), lambda bb, tt: (bb, tt, 0))],
        out_specs=pl.BlockSpec(
            (8, 8, 1, 8, IDX_MINOR), lambda bb, tt: (tt, 0, bb, 0, 0)),
        out_shape=jax.ShapeDtypeStruct((SEQ, 8, NW, 8, IDX_MINOR), jnp.float32),
    )(x3)


def kernel(tokens, table):
    idx = jnp.asarray(tokens, jnp.int32).reshape(NW * ROWS_PER_W, IDX_MINOR)
    flat = _sc_gather(jnp.asarray(table, jnp.float32), idx)
    out5 = _tc_finish(flat)
    return out5.transpose(2, 4, 0, 1, 3).reshape(BATCH, SEQ, D)


# speedup vs baseline: 1.8416x; 1.8416x over previous
import functools

import jax
import jax.numpy as jnp
from jax import lax
from jax.experimental import pallas as pl
from jax.experimental.pallas import tpu as pltpu
from jax.experimental.pallas import tpu_sc as plsc

D = 64
SCALE = 8.0

NC = 2
NS = 16
NW = NC * NS

BATCH = 4096
SEQ = 200
BW = BATCH // NW
TT = SEQ // 8
NBUF = 4
L = 16


@functools.partial(
    pl.kernel,
    out_type=jax.ShapeDtypeStruct((SEQ, D // 8, NW, 8 * BW), jnp.float32),
    mesh=plsc.VectorSubcoreMesh(core_axis_name="c", subcore_axis_name="s"),
    compiler_params=pltpu.CompilerParams(
        use_tc_tiling_on_sc=False, needs_layout_passes=False),
    scratch_types=[
        pltpu.VMEM((TT, 8, BW), jnp.int32),
        pltpu.VMEM((BW, D), jnp.float32),
        pltpu.VMEM((BW, D), jnp.float32),
        pltpu.VMEM((BW, D), jnp.float32),
        pltpu.VMEM((BW, D), jnp.float32),
        pltpu.VMEM((D * BW,), jnp.float32),
        pltpu.VMEM((D * BW,), jnp.float32),
        pltpu.VMEM((D * BW,), jnp.float32),
        pltpu.VMEM((D * BW,), jnp.float32),
        pltpu.SemaphoreType.DMA,
        pltpu.SemaphoreType.DMA,
        pltpu.SemaphoreType.DMA,
        pltpu.SemaphoreType.DMA,
        pltpu.SemaphoreType.DMA,
        pltpu.SemaphoreType.DMA,
        pltpu.SemaphoreType.DMA,
        pltpu.SemaphoreType.DMA,
    ],
)
def _sc_embed(tok_hbm, table_hbm, out_hbm,
              idx_v, s0, s1, s2, s3, t0, t1, t2, t3,
              g0, g1, g2, g3, w0, w1, w2, w3):
    slab = (s0, s1, s2, s3)
    slabt = (t0, t1, t2, t3)
    gsem = (g0, g1, g2, g3)
    wsem = (w0, w1, w2, w3)

    wid = lax.axis_index("s") * NC + lax.axis_index("c")

    pltpu.sync_copy(tok_hbm.at[pl.ds(0, TT), wid], idx_v)

    lane = lax.iota(jnp.int32, L)
    diag = [jnp.bitwise_and(lane + s, L - 1) for s in range(L)]

    def fire(r, b):
        pltpu.async_copy(
            table_hbm.at[idx_v.at[r // 8, r % 8]], slab[b], gsem[b])

    def drain(r, b):
        pltpu.make_async_copy(
            table_hbm.at[idx_v.at[r // 8, r % 8]], slab[b], gsem[b]).wait()

    def write(r, b):
        for dt in range(D // 8):
            pltpu.async_copy(
                slabt[b].at[pl.ds(dt * 8 * BW, 8 * BW)],
                out_hbm.at[r, dt, wid], wsem[b])

    def wait_write(r, b):
        for dt in range(D // 8):
            pltpu.make_async_copy(
                slabt[b].at[pl.ds(dt * 8 * BW, 8 * BW)],
                out_hbm.at[r, dt, wid], wsem[b]).wait()

    def transpose_scale(b):
        src, dst = slab[b], slabt[b]

        @plsc.parallel_loop(0, BW // L, unroll=1)
        def _(bg):
            bvec = bg * L + lane
            for dg in range(D // L):
                for s in range(L):
                    dvec = dg * L + diag[s]
                    v = plsc.load_gather(src, [bvec, dvec])
                    plsc.store_scatter(dst, [dvec * BW + bvec], v * SCALE)

    for r in range(NBUF - 1):
        fire(r, r)

    def step(p, carry):
        for b in range(NBUF):
            r = p * NBUF + b
            drain(r, b)

            @pl.when(r >= NBUF)
            def _():
                wait_write(r - NBUF, b)

            transpose_scale(b)
            write(r, b)
            fr = r + NBUF - 1

            @pl.when(fr < SEQ)
            def _():
                fire(fr, (b + NBUF - 1) % NBUF)
        return carry

    lax.fori_loop(0, SEQ // NBUF, step, 0)

    for b in range(NBUF):
        wait_write(SEQ - NBUF + b, b)


def kernel(tokens, table):
    tok4 = jnp.transpose(
        jnp.transpose(jnp.asarray(tokens, jnp.int32)).reshape(TT, 8, NW, BW),
        (0, 2, 1, 3))
    out5 = _sc_embed(tok4, jnp.asarray(table, jnp.float32))
    return (out5.reshape(SEQ, D // 8, NW, 8, BW)
            .transpose(2, 4, 0, 1, 3)
            .reshape(BATCH, SEQ, D))
